# baseline (speedup 1.0000x reference)
import functools

import jax
import jax.numpy as jnp
from jax import lax
from jax.experimental import pallas as pl
from jax.experimental.pallas import tpu as pltpu
from jax.experimental.pallas import tpu_sc as plsc

B = 16384
F = 32
L = 16
NC = 2
NS = 16
NW = NC * NS
BPW = B // NW


def _sc_body(time_hbm, item_hbm, tf_hbm, if_hbm, out_hbm,
             tidx_v, iidx_v, trows_v, irows_v, out_v, sem_t, sem_i):
    wid = lax.axis_index("s") * NC + lax.axis_index("c")
    base = wid * BPW

    pltpu.sync_copy(time_hbm.at[pl.ds(base, BPW)], tidx_v)
    pltpu.sync_copy(item_hbm.at[pl.ds(base, BPW)], iidx_v)

    ct = pltpu.async_copy(tf_hbm.at[tidx_v], trows_v, sem_t)
    ci = pltpu.async_copy(if_hbm.at[iidx_v], irows_v, sem_i)
    ct.wait()
    ci.wait()

    lane = lax.iota(jnp.int32, L)

    def group(g, carry):
        acc = jnp.zeros((L,), jnp.float32)
        for u in range(L):
            r = g * L + u
            t0 = trows_v[r, pl.ds(0, L)]
            t1 = trows_v[r, pl.ds(L, L)]
            i0 = irows_v[r, pl.ds(0, L)]
            i1 = irows_v[r, pl.ds(L, L)]
            p = t0 * i0 + t1 * i1
            acc = jnp.where(lane == u, jnp.sum(p), acc)
        out_v[pl.ds(g * L, L)] = acc
        return carry

    lax.fori_loop(0, BPW // L, group, 0)

    pltpu.sync_copy(out_v, out_hbm.at[pl.ds(base, BPW)])


@jax.jit
def _run(time, item, time_factor, item_factor):
    kern = pl.kernel(
        _sc_body,
        out_type=jax.ShapeDtypeStruct((B,), jnp.float32),
        mesh=plsc.VectorSubcoreMesh(core_axis_name="c", subcore_axis_name="s"),
        compiler_params=pltpu.CompilerParams(
            needs_layout_passes=False, use_tc_tiling_on_sc=False),
        scratch_types=[
            pltpu.VMEM((BPW,), jnp.int32),
            pltpu.VMEM((BPW,), jnp.int32),
            pltpu.VMEM((BPW, F), jnp.float32),
            pltpu.VMEM((BPW, F), jnp.float32),
            pltpu.VMEM((BPW,), jnp.float32),
            pltpu.SemaphoreType.DMA,
            pltpu.SemaphoreType.DMA,
        ],
    )
    return kern(time, item, time_factor, item_factor)


def kernel(time, item, time_factor, item_factor, lag_factor):
    del lag_factor
    return _run(time.astype(jnp.int32), item.astype(jnp.int32),
                time_factor, item_factor)
